# baseline (speedup 1.0000x reference)
import functools

import jax
import jax.numpy as jnp
from jax import lax
from jax.experimental import pallas as pl
from jax.experimental.pallas import tpu as pltpu
from jax.experimental.pallas import tpu_sc as plsc

NC, NS = 2, 16
NW = NC * NS
LANE = 128
NB = 8
ND = NW // NB


def _make_sc_gather_t(b, v, d):
    nchunk_total = d // LANE
    per = nchunk_total // ND
    extra = nchunk_total - per * ND
    assert b == NB * LANE

    mesh = plsc.VectorSubcoreMesh(core_axis_name="c", subcore_axis_name="s")

    @functools.partial(
        pl.kernel,
        out_type=jax.ShapeDtypeStruct((d, b), jnp.float32),
        mesh=mesh,
        scratch_types=[
            pltpu.VMEM((LANE,), jnp.int32),
            pltpu.VMEM((2, LANE, LANE), jnp.float32),
            pltpu.VMEM((2, LANE, LANE), jnp.float32),
            pltpu.SemaphoreType.DMA((2,)),
            pltpu.SemaphoreType.DMA((2,)),
        ],
        compiler_params=pltpu.CompilerParams(needs_layout_passes=False),
    )
    def gather(e_hbm, table_hbm, out_hbm, idx_v, gbufs, tbufs, gsem, ssem):
        wid = lax.axis_index("s") * NC + lax.axis_index("c")
        dblk = wid // NB
        bblk = wid % NB
        n = jnp.where(dblk < extra, per + 1, per)
        c0 = dblk * per + jnp.minimum(dblk, extra)
        pltpu.sync_copy(e_hbm.at[pl.ds(bblk * LANE, LANE)], idx_v)
        lanes = lax.iota(jnp.int32, 16)

        def gcopy(lc, slot):
            return pltpu.make_async_copy(
                table_hbm.at[idx_v, pl.ds((c0 + lc) * LANE, LANE)],
                gbufs.at[slot],
                gsem.at[slot],
            )

        def scopy(lc, slot):
            return pltpu.make_async_copy(
                tbufs.at[slot],
                out_hbm.at[pl.ds((c0 + lc) * LANE, LANE), pl.ds(bblk * LANE, LANE)],
                ssem.at[slot],
            )

        def transpose(slot):
            def tj(j, _):
                cols = lanes * 0 + j
                for k in range(8):
                    vec = plsc.load_gather(
                        gbufs.at[slot], [lanes + 16 * k, cols]
                    )
                    tbufs[slot, j, pl.ds(16 * k, 16)] = vec
                return ()

            lax.fori_loop(0, LANE, tj, ())

        gcopy(0, 0).start()

        def body(lc, _):
            slot = lax.rem(lc, 2)
            oslot = lax.rem(lc + 1, 2)
            gcopy(lc, slot).wait()

            @pl.when(lc + 1 < n)
            def _():
                gcopy(lc + 1, oslot).start()

            @pl.when(lc >= 2)
            def _():
                scopy(lc - 2, slot).wait()

            @pl.when(slot == 0)
            def _():
                transpose(0)

            @pl.when(slot == 1)
            def _():
                transpose(1)

            scopy(lc, slot).start()
            return ()

        lax.fori_loop(0, n, body, ())
        scopy(n - 1, lax.rem(n - 1, 2)).wait()
        scopy(n - 2, lax.rem(n, 2)).wait()

    return gather


def _make_tc_tail(b, v, d):
    dal = (d // LANE) * LANE
    tail = d - dal
    tailblk = dal // LANE

    def tail_kernel(e_ref, wtail_ref, _, o_ref):
        e = e_ref[:]
        onehot_t = (
            lax.broadcasted_iota(jnp.int32, (v, b), 0) == e[None, :]
        ).astype(jnp.float32)
        res = lax.dot_general(
            wtail_ref[...],
            onehot_t,
            (((0,), (0,)), ((), ())),
            preferred_element_type=jnp.float32,
        )
        o_ref[...] = res[:tail, :]

    return pl.pallas_call(
        tail_kernel,
        grid=(1,),
        out_shape=jax.ShapeDtypeStruct((d, b), jnp.float32),
        in_specs=[
            pl.BlockSpec((b,), lambda i: (0,)),
            pl.BlockSpec((v, LANE), lambda i: (0, tailblk)),
            pl.BlockSpec(memory_space=pl.ANY),
        ],
        out_specs=pl.BlockSpec((tail, b), lambda i: (dal // tail, 0)),
        input_output_aliases={2: 0},
    )


def kernel(e, weight):
    b = e.shape[0]
    v, d = weight.shape
    ei = e.astype(jnp.int32)
    tout = _make_sc_gather_t(b, v, d)(ei, weight)
    tout = _make_tc_tail(b, v, d)(ei, weight, tout)
    return tout.T.reshape(b, 1, d)


# speedup vs baseline: 1.8318x; 1.8318x over previous
import functools

import jax
import jax.numpy as jnp
from jax import lax
from jax.experimental import pallas as pl
from jax.experimental.pallas import tpu as pltpu
from jax.experimental.pallas import tpu_sc as plsc

NC, NS = 2, 16
NW = NC * NS
LANE = 128
NB = 8
ND = NW // NB


def _make_sc_gather_t(b, v, d):
    nchunk_total = d // LANE
    per = nchunk_total // ND
    extra = nchunk_total - per * ND
    assert b == NB * LANE

    mesh = plsc.VectorSubcoreMesh(core_axis_name="c", subcore_axis_name="s")

    @functools.partial(
        pl.kernel,
        out_type=jax.ShapeDtypeStruct((d, b), jnp.float32),
        mesh=mesh,
        scratch_types=[
            pltpu.VMEM((LANE,), jnp.int32),
            pltpu.VMEM((2, LANE, LANE), jnp.float32),
            pltpu.VMEM((2, LANE, LANE), jnp.float32),
            pltpu.SemaphoreType.DMA((2,)),
            pltpu.SemaphoreType.DMA((2,)),
        ],
        compiler_params=pltpu.CompilerParams(needs_layout_passes=False),
    )
    def gather(e_hbm, table_hbm, out_hbm, idx_v, gbufs, tbufs, gsem, ssem):
        wid = lax.axis_index("s") * NC + lax.axis_index("c")
        dblk = wid // NB
        bblk = wid % NB
        n = jnp.where(dblk < extra, per + 1, per)
        c0 = dblk * per + jnp.minimum(dblk, extra)
        pltpu.sync_copy(e_hbm.at[pl.ds(bblk * LANE, LANE)], idx_v)
        lanes = lax.iota(jnp.int32, 16)

        def gcopy(lc, slot):
            return pltpu.make_async_copy(
                table_hbm.at[idx_v, pl.ds((c0 + lc) * LANE, LANE)],
                gbufs.at[slot],
                gsem.at[slot],
            )

        def scopy(lc, slot):
            return pltpu.make_async_copy(
                tbufs.at[slot],
                out_hbm.at[pl.ds((c0 + lc) * LANE, LANE), pl.ds(bblk * LANE, LANE)],
                ssem.at[slot],
            )

        def transpose(slot):
            @plsc.parallel_loop(0, LANE, step=1, unroll=8)
            def _(j):
                cols = lanes * 0 + j
                for k in range(8):
                    vec = plsc.load_gather(
                        gbufs.at[slot], [lanes + 16 * k, cols]
                    )
                    tbufs[slot, j, pl.ds(16 * k, 16)] = vec

        gcopy(0, 0).start()

        def body(lc, _):
            slot = lax.rem(lc, 2)
            oslot = lax.rem(lc + 1, 2)
            gcopy(lc, slot).wait()

            @pl.when(lc + 1 < n)
            def _():
                gcopy(lc + 1, oslot).start()

            @pl.when(lc >= 2)
            def _():
                scopy(lc - 2, slot).wait()

            @pl.when(slot == 0)
            def _():
                transpose(0)

            @pl.when(slot == 1)
            def _():
                transpose(1)

            scopy(lc, slot).start()
            return ()

        lax.fori_loop(0, n, body, ())
        scopy(n - 1, lax.rem(n - 1, 2)).wait()
        scopy(n - 2, lax.rem(n, 2)).wait()

    return gather


def _make_tc_tail(b, v, d):
    dal = (d // LANE) * LANE
    tail = d - dal
    tailblk = dal // LANE

    def tail_kernel(e_ref, wtail_ref, _, o_ref):
        e = e_ref[:]
        onehot_t = (
            lax.broadcasted_iota(jnp.int32, (v, b), 0) == e[None, :]
        ).astype(jnp.float32)
        res = lax.dot_general(
            wtail_ref[...],
            onehot_t,
            (((0,), (0,)), ((), ())),
            preferred_element_type=jnp.float32,
        )
        o_ref[...] = res[:tail, :]

    return pl.pallas_call(
        tail_kernel,
        grid=(1,),
        out_shape=jax.ShapeDtypeStruct((d, b), jnp.float32),
        in_specs=[
            pl.BlockSpec((b,), lambda i: (0,)),
            pl.BlockSpec((v, LANE), lambda i: (0, tailblk)),
            pl.BlockSpec(memory_space=pl.ANY),
        ],
        out_specs=pl.BlockSpec((tail, b), lambda i: (dal // tail, 0)),
        input_output_aliases={2: 0},
    )


def kernel(e, weight):
    b = e.shape[0]
    v, d = weight.shape
    ei = e.astype(jnp.int32)
    tout = _make_sc_gather_t(b, v, d)(ei, weight)
    tout = _make_tc_tail(b, v, d)(ei, weight, tout)
    return tout.T.reshape(b, 1, d)


# speedup vs baseline: 4.3129x; 2.3544x over previous
import functools

import jax
import jax.numpy as jnp
from jax import lax
from jax.experimental import pallas as pl
from jax.experimental.pallas import tpu as pltpu
from jax.experimental.pallas import tpu_sc as plsc

NC, NS = 2, 16
NW = NC * NS
LANE = 128
CW = 1408


def _make_sc_gather(b, v, d):
    rpw = b // NW
    dal = (d // LANE) * LANE
    nb = dal // CW
    assert b % NW == 0 and rpw % 8 == 0 and dal % CW == 0 and nb >= 3

    mesh = plsc.VectorSubcoreMesh(core_axis_name="c", subcore_axis_name="s")

    @functools.partial(
        pl.kernel,
        out_type=jax.ShapeDtypeStruct((b, d), jnp.float32),
        mesh=mesh,
        scratch_types=[
            pltpu.VMEM((rpw,), jnp.int32),
            pltpu.VMEM((2, rpw, CW), jnp.float32),
            pltpu.SemaphoreType.DMA((2,)),
            pltpu.SemaphoreType.DMA((2,)),
        ],
    )
    def gather(e_hbm, table_hbm, out_hbm, idx_v, bufs, gsem, ssem):
        wid = lax.axis_index("s") * NC + lax.axis_index("c")
        base = wid * rpw
        pltpu.sync_copy(e_hbm.at[pl.ds(base, rpw)], idx_v)

        def gcopy(c, slot):
            return pltpu.make_async_copy(
                table_hbm.at[idx_v, pl.ds(c * CW, CW)],
                bufs.at[slot],
                gsem.at[slot],
            )

        def scopy(c, slot):
            return pltpu.make_async_copy(
                bufs.at[slot],
                out_hbm.at[pl.ds(base, rpw), pl.ds(c * CW, CW)],
                ssem.at[slot],
            )

        gcopy(0, 0).start()
        gcopy(1, 1).start()
        gcopy(0, 0).wait()
        scopy(0, 0).start()

        def body(c, _):
            slot = lax.rem(c, 2)
            nslot = lax.rem(c + 1, 2)
            gcopy(c, slot).wait()
            scopy(c, slot).start()
            scopy(c - 1, nslot).wait()
            gcopy(c + 1, nslot).start()
            return ()

        lax.fori_loop(1, nb - 1, body, ())

        last = nb - 1
        lslot = lax.rem(last, 2)
        gcopy(last, lslot).wait()
        scopy(last, lslot).start()
        scopy(last - 1, lax.rem(last + 1, 2)).wait()
        scopy(last, lslot).wait()

    return gather


def _make_tc_tail(b, v, d):
    dal = (d // LANE) * LANE
    tailblk = dal // LANE

    def tail_kernel(e_ref, wtail_ref, _, o_ref):
        e = e_ref[:]
        onehot = (
            e[:, None] == lax.broadcasted_iota(jnp.int32, (b, v), 1)
        ).astype(jnp.float32)
        o_ref[...] = jnp.dot(
            onehot, wtail_ref[...], preferred_element_type=jnp.float32
        )

    return pl.pallas_call(
        tail_kernel,
        grid=(1,),
        out_shape=jax.ShapeDtypeStruct((b, d), jnp.float32),
        in_specs=[
            pl.BlockSpec((b,), lambda i: (0,)),
            pl.BlockSpec((v, LANE), lambda i: (0, tailblk)),
            pl.BlockSpec(memory_space=pl.ANY),
        ],
        out_specs=pl.BlockSpec((b, LANE), lambda i: (0, tailblk)),
        input_output_aliases={2: 0},
    )


def kernel(e, weight):
    b = e.shape[0]
    v, d = weight.shape
    ei = e.astype(jnp.int32)
    out = _make_sc_gather(b, v, d)(ei, weight)
    out = _make_tc_tail(b, v, d)(ei, weight, out)
    return out.reshape(b, 1, d)
